# Initial kernel scaffold; baseline (speedup 1.0000x reference)
import jax
import jax.numpy as jnp
from jax.experimental import pallas as pl


def kernel(z_e_sample, embeddings, alpha_som):
    raise NotImplementedError("write your pallas kernel here")


# speedup vs baseline: 2.8816x; 2.8816x over previous
import functools

import jax
import jax.numpy as jnp
from jax import lax
from jax.experimental import pallas as pl
from jax.experimental.pallas import tpu as pltpu
from jax.experimental.pallas import tpu_sc as plsc

SOM_H, SOM_W = 64, 64
N_NODES = SOM_H * SOM_W
LATENT = 256
N_ROWS = 8192
TC_BLOCK = 512

_EPS_F32 = 1.1920929e-07


def _tc_body(alpha_ref, z_ref, e_ref, dist_ref, bmu_ref, q_ref):
    zb = z_ref[...]
    eb = e_ref[...]
    z2 = jnp.sum(zb * zb, axis=1, keepdims=True)
    e2 = jnp.sum(eb * eb, axis=1)[None, :]
    dot = lax.dot_general(zb, eb, (((1,), (1,)), ((), ())),
                          preferred_element_type=jnp.float32)
    d = z2 + e2 - 2.0 * dot
    d = jnp.maximum(d, 0.0)
    dist_ref[...] = d

    dmin = jnp.min(d, axis=1, keepdims=True)
    ids = lax.broadcasted_iota(jnp.int32, d.shape, 1)
    bmu_ref[...] = jnp.min(
        jnp.where(d == dmin, ids, jnp.int32(N_NODES)), axis=1)

    af = alpha_ref[0, 0]
    ex = (af + 1.0) * 0.5
    qn = 1.0 / (1.0 + d / af)

    def _write_q(qu):
        s = jnp.sum(qu, axis=1, keepdims=True)
        q_ref[...] = qu / s + _EPS_F32

    @pl.when(ex == 1.0)
    def _():
        _write_q(qn)

    @pl.when(ex != 1.0)
    def _():
        _write_q(jnp.exp(jnp.log(qn) * ex))


def _tc_call(af, z, e):
    grid = (N_ROWS // TC_BLOCK,)
    return pl.pallas_call(
        _tc_body,
        grid=grid,
        in_specs=[
            pl.BlockSpec(memory_space=pltpu.SMEM),
            pl.BlockSpec((TC_BLOCK, LATENT), lambda i: (i, 0)),
            pl.BlockSpec((N_NODES, LATENT), lambda i: (0, 0)),
        ],
        out_specs=[
            pl.BlockSpec((TC_BLOCK, N_NODES), lambda i: (i, 0)),
            pl.BlockSpec((TC_BLOCK,), lambda i: (i,)),
            pl.BlockSpec((TC_BLOCK, N_NODES), lambda i: (i, 0)),
        ],
        out_shape=[
            jax.ShapeDtypeStruct((N_ROWS, N_NODES), jnp.float32),
            jax.ShapeDtypeStruct((N_ROWS,), jnp.int32),
            jax.ShapeDtypeStruct((N_ROWS, N_NODES), jnp.float32),
        ],
    )(af, z, e)


def _sc_gather(embeddings, bmu):
    info = plsc.get_sparse_core_info()
    nc, ns = info.num_cores, info.num_subcores
    nw = nc * ns
    rows_w = N_ROWS // nw
    zq_chunk = 128
    nbr_rows = 16

    mesh = plsc.VectorSubcoreMesh(core_axis_name="c", subcore_axis_name="s")

    @functools.partial(
        pl.kernel,
        mesh=mesh,
        out_type=[
            jax.ShapeDtypeStruct((N_ROWS, LATENT), jnp.float32),
            jax.ShapeDtypeStruct((N_ROWS * 5, LATENT), jnp.float32),
        ],
        scratch_types=[
            pltpu.VMEM((rows_w,), jnp.int32),
            pltpu.VMEM((nbr_rows * 5,), jnp.int32),
            pltpu.VMEM((zq_chunk, LATENT), jnp.float32),
            pltpu.VMEM((nbr_rows * 5, LATENT), jnp.float32),
            pltpu.SemaphoreType.DMA,
        ],
    )
    def k(emb_hbm, bmu_hbm, zq_hbm, nbr_hbm, bmu_v, idx_v, zrows_v,
          nrows_v, sem):
        wid = lax.axis_index("s") * nc + lax.axis_index("c")
        base = wid * rows_w
        pltpu.sync_copy(bmu_hbm.at[pl.ds(base, rows_w)], bmu_v)

        for c in range(rows_w // zq_chunk):
            pltpu.async_copy(
                emb_hbm.at[bmu_v.at[pl.ds(c * zq_chunk, zq_chunk)]],
                zrows_v, sem).wait()
            pltpu.sync_copy(
                zrows_v, zq_hbm.at[pl.ds(base + c * zq_chunk, zq_chunk)])

        lane = lax.iota(jnp.int32, 16)
        dnums = lax.GatherDimensionNumbers(
            offset_dims=(), collapsed_slice_dims=(0,), start_index_map=(0,))
        for c in range(rows_w // nbr_rows):
            v = bmu_v[pl.ds(c * nbr_rows, 16)]
            for t in range(5):
                p = t * 16 + lane
                i_rel = lax.shift_right_logical(p * 52429, 18)
                j_rel = p - i_rel * 5
                vi = lax.gather(
                    v, i_rel[:, None], dnums, (1,),
                    mode=lax.GatherScatterMode.PROMISE_IN_BOUNDS)
                k1 = lax.shift_right_logical(vi, 6)
                k2 = vi & 63
                dk1 = (jnp.where(j_rel == 1, -1, 0)
                       + jnp.where(j_rel == 2, 1, 0))
                dk2 = (jnp.where(j_rel == 3, 1, 0)
                       + jnp.where(j_rel == 4, -1, 0))
                nv = (lax.shift_left((k1 + dk1 + 64) & 63, 6)
                      | ((k2 + dk2 + 64) & 63))
                idx_v[pl.ds(t * 16, 16)] = nv
            pltpu.async_copy(emb_hbm.at[idx_v], nrows_v, sem).wait()
            pltpu.sync_copy(
                nrows_v,
                nbr_hbm.at[pl.ds((base + c * nbr_rows) * 5, nbr_rows * 5)])

    return k(embeddings, bmu)


def kernel(z_e_sample, embeddings, alpha_som):
    af = jnp.asarray(alpha_som, jnp.float32).reshape(1, 1)
    z_dist, bmu, q = _tc_call(af, z_e_sample, embeddings)
    z_q, nbr_flat = _sc_gather(embeddings, bmu)
    z_q_neighbors = nbr_flat.reshape(N_ROWS, 5, LATENT)
    return (z_dist, bmu, z_q, z_q_neighbors, q)


# speedup vs baseline: 3.0680x; 1.0647x over previous
import functools

import jax
import jax.numpy as jnp
from jax import lax
from jax.experimental import pallas as pl
from jax.experimental.pallas import tpu as pltpu
from jax.experimental.pallas import tpu_sc as plsc

SOM_H, SOM_W = 64, 64
N_NODES = SOM_H * SOM_W
LATENT = 256
N_ROWS = 8192
TC_BLOCK = 512

_EPS_F32 = 1.1920929e-07


def _tc_body(alpha_ref, z_ref, e_ref, dist_ref, bmu_ref, q_ref, e2_ref):
    zb = z_ref[...]

    @pl.when(pl.program_id(0) == 0)
    def _():
        eb = e_ref[...]
        e2_ref[...] = jnp.sum(eb * eb, axis=1)[None, :]

    z2 = jnp.sum(zb * zb, axis=1, keepdims=True)
    e2 = e2_ref[...]
    dot = lax.dot_general(zb, e_ref[...], (((1,), (1,)), ((), ())),
                          preferred_element_type=jnp.float32)
    d = z2 + e2 - 2.0 * dot
    d = jnp.maximum(d, 0.0)
    dist_ref[...] = d
    bmu_ref[...] = jnp.argmin(d, axis=1).astype(jnp.int32)

    af = alpha_ref[0, 0]
    ex = (af + 1.0) * 0.5

    @pl.when(ex == 1.0)
    def _():
        s = jnp.sum(1.0 / (1.0 + d / af), axis=1, keepdims=True)
        q_ref[...] = (1.0 / (1.0 + d / af)) / s + _EPS_F32

    @pl.when(ex != 1.0)
    def _():
        qn = jnp.exp(jnp.log(1.0 / (1.0 + d / af)) * ex)
        s = jnp.sum(qn, axis=1, keepdims=True)
        q_ref[...] = qn / s + _EPS_F32


def _tc_call(af, z, e):
    grid = (N_ROWS // TC_BLOCK,)
    return pl.pallas_call(
        _tc_body,
        grid=grid,
        in_specs=[
            pl.BlockSpec(memory_space=pltpu.SMEM),
            pl.BlockSpec((TC_BLOCK, LATENT), lambda i: (i, 0)),
            pl.BlockSpec((N_NODES, LATENT), lambda i: (0, 0)),
        ],
        out_specs=[
            pl.BlockSpec((TC_BLOCK, N_NODES), lambda i: (i, 0)),
            pl.BlockSpec((TC_BLOCK,), lambda i: (i,)),
            pl.BlockSpec((TC_BLOCK, N_NODES), lambda i: (i, 0)),
        ],
        out_shape=[
            jax.ShapeDtypeStruct((N_ROWS, N_NODES), jnp.float32),
            jax.ShapeDtypeStruct((N_ROWS,), jnp.int32),
            jax.ShapeDtypeStruct((N_ROWS, N_NODES), jnp.float32),
        ],
        scratch_shapes=[pltpu.VMEM((1, N_NODES), jnp.float32)],
    )(af, z, e)


def _sc_gather(embeddings, bmu):
    info = plsc.get_sparse_core_info()
    nc, ns = info.num_cores, info.num_subcores
    nw = nc * ns
    rows_w = N_ROWS // nw
    zq_chunk = 128
    nbr_rows = 16

    mesh = plsc.VectorSubcoreMesh(core_axis_name="c", subcore_axis_name="s")

    @functools.partial(
        pl.kernel,
        mesh=mesh,
        out_type=[
            jax.ShapeDtypeStruct((N_ROWS, LATENT), jnp.float32),
            jax.ShapeDtypeStruct((N_ROWS * 5, LATENT), jnp.float32),
        ],
        scratch_types=[
            pltpu.VMEM((rows_w,), jnp.int32),
            pltpu.VMEM((nbr_rows * 5,), jnp.int32),
            pltpu.VMEM((zq_chunk, LATENT), jnp.float32),
            pltpu.VMEM((nbr_rows * 5, LATENT), jnp.float32),
            pltpu.SemaphoreType.DMA,
        ],
    )
    def k(emb_hbm, bmu_hbm, zq_hbm, nbr_hbm, bmu_v, idx_v, zrows_v,
          nrows_v, sem):
        wid = lax.axis_index("s") * nc + lax.axis_index("c")
        base = wid * rows_w
        pltpu.sync_copy(bmu_hbm.at[pl.ds(base, rows_w)], bmu_v)

        for c in range(rows_w // zq_chunk):
            pltpu.async_copy(
                emb_hbm.at[bmu_v.at[pl.ds(c * zq_chunk, zq_chunk)]],
                zrows_v, sem).wait()
            pltpu.sync_copy(
                zrows_v, zq_hbm.at[pl.ds(base + c * zq_chunk, zq_chunk)])

        lane = lax.iota(jnp.int32, 16)
        dnums = lax.GatherDimensionNumbers(
            offset_dims=(), collapsed_slice_dims=(0,), start_index_map=(0,))
        for c in range(rows_w // nbr_rows):
            v = bmu_v[pl.ds(c * nbr_rows, 16)]
            for t in range(5):
                p = t * 16 + lane
                i_rel = lax.shift_right_logical(p * 52429, 18)
                j_rel = p - i_rel * 5
                vi = lax.gather(
                    v, i_rel[:, None], dnums, (1,),
                    mode=lax.GatherScatterMode.PROMISE_IN_BOUNDS)
                k1 = lax.shift_right_logical(vi, 6)
                k2 = vi & 63
                dk1 = (jnp.where(j_rel == 1, -1, 0)
                       + jnp.where(j_rel == 2, 1, 0))
                dk2 = (jnp.where(j_rel == 3, 1, 0)
                       + jnp.where(j_rel == 4, -1, 0))
                nv = (lax.shift_left((k1 + dk1 + 64) & 63, 6)
                      | ((k2 + dk2 + 64) & 63))
                idx_v[pl.ds(t * 16, 16)] = nv
            pltpu.async_copy(emb_hbm.at[idx_v], nrows_v, sem).wait()
            pltpu.sync_copy(
                nrows_v,
                nbr_hbm.at[pl.ds((base + c * nbr_rows) * 5, nbr_rows * 5)])

    return k(embeddings, bmu)


def kernel(z_e_sample, embeddings, alpha_som):
    af = jnp.asarray(alpha_som, jnp.float32).reshape(1, 1)
    z_dist, bmu, q = _tc_call(af, z_e_sample, embeddings)
    z_q, nbr_flat = _sc_gather(embeddings, bmu)
    z_q_neighbors = nbr_flat.reshape(N_ROWS, 5, LATENT)
    return (z_dist, bmu, z_q, z_q_neighbors, q)
